# baseline (speedup 1.0000x reference)
import functools

import jax
import jax.numpy as jnp
from jax import lax
from jax.experimental import pallas as pl
from jax.experimental.pallas import tpu as pltpu
from jax.experimental.pallas import tpu_sc as plsc

B = 16384
F = 32
NC = 2
NS = 16
NW = NC * NS
BPW = B // NW
CHUNK = 128
NCH = BPW // CHUNK
GRP = CHUNK // 16

BLK = 65536
QTR = BLK // 4

N_U = 1000000
N_I = 100000
UBLKS = (N_U + BLK - 1) // BLK
HBLKS = UBLKS // 2
SPLIT = HBLKS * BLK
HROWS = HBLKS * QTR


def _repack_body(src_ref, dst_ref):
    for s in range(BLK // 512):
        z = jnp.concatenate(
            [src_ref[:, pl.ds(512 * s + 128 * g, 128)] for g in range(4)],
            axis=0)
        dst_ref[pl.ds(s * 128, 128), :] = jnp.transpose(z)


def _repack(table_t, blk0, nblk):
    return pl.pallas_call(
        _repack_body,
        grid=(nblk,),
        in_specs=[pl.BlockSpec((F, BLK), lambda i: (0, i + blk0))],
        out_specs=pl.BlockSpec((QTR, 128), lambda i: (i, 0)),
        out_shape=jax.ShapeDtypeStruct((nblk * QTR, 128), jnp.float32),
    )(table_t)


def _pack_row(u):
    return lax.shift_left(
        lax.shift_right_logical(u, 9), 7) + jnp.bitwise_and(u, 127)


def _pack_col(u):
    return lax.shift_left(
        jnp.bitwise_and(lax.shift_right_logical(u, 7), 3), 5)


_mesh = plsc.VectorSubcoreMesh(core_axis_name="c", subcore_axis_name="s")
_sc_params = pltpu.CompilerParams(needs_layout_passes=False)


def _dot_chunks(uidx, iidx, urow, irow, uf_hbm, if_hbm, ubuf, ibuf,
                sems_u, sems_i, emit):
    def fire(j):
        p = j % 2
        cu = pltpu.async_copy(uf_hbm.at[urow.at[j]], ubuf.at[p], sems_u[p])
        ci = pltpu.async_copy(if_hbm.at[irow.at[j]], ibuf.at[p], sems_i[p])
        return cu, ci

    pending = fire(0)
    for j in range(NCH):
        nxt = fire(j + 1) if j + 1 < NCH else None
        cu, ci = pending
        cu.wait()
        ci.wait()
        pending = nxt
        p = j % 2

        def body(g, carry):
            rows = g * 16 + lax.iota(jnp.int32, 16)
            s = pl.ds(g * 16, 16)
            ucol = _pack_col(uidx[j, s])
            icol = _pack_col(iidx[j, s])
            acc = jnp.zeros((16,), jnp.float32)
            for f in range(F):
                gu = plsc.load_gather(ubuf.at[p], [rows, ucol + f])
                gi = plsc.load_gather(ibuf.at[p], [rows, icol + f])
                acc = acc + gu * gi
            emit(j, g, s, acc)
            return carry

        lax.fori_loop(0, GRP, body, 0)


_scratch = [
    pltpu.VMEM((NCH, CHUNK), jnp.int32),
    pltpu.VMEM((NCH, CHUNK), jnp.int32),
    pltpu.VMEM((NCH, CHUNK), jnp.int32),
    pltpu.VMEM((NCH, CHUNK), jnp.int32),
    pltpu.VMEM((2, CHUNK, 128), jnp.float32),
    pltpu.VMEM((2, CHUNK, 128), jnp.float32),
    pltpu.VMEM((BPW,), jnp.float32),
    pltpu.SemaphoreType.DMA,
    pltpu.SemaphoreType.DMA,
    pltpu.SemaphoreType.DMA,
    pltpu.SemaphoreType.DMA,
]


@functools.partial(
    pl.kernel, mesh=_mesh,
    out_type=jax.ShapeDtypeStruct((B,), jnp.float32),
    compiler_params=_sc_params, scratch_types=_scratch,
)
def _mf_pass1(user_hbm, item_hbm, uf0_hbm, if_hbm, out_hbm,
              uidx, iidx, urow, irow, ubuf, ibuf, outv,
              sem_u0, sem_u1, sem_i0, sem_i1):
    wid = lax.axis_index("s") * NC + lax.axis_index("c")
    base = wid * BPW

    idx_copies = []
    for j in range(NCH):
        idx_copies.append(pltpu.async_copy(
            user_hbm.at[pl.ds(base + j * CHUNK, CHUNK)], uidx.at[j], sem_u0))
        idx_copies.append(pltpu.async_copy(
            item_hbm.at[pl.ds(base + j * CHUNK, CHUNK)], iidx.at[j], sem_i0))
    for c in idx_copies:
        c.wait()
    for j in range(NCH):
        for g in range(GRP):
            s = pl.ds(g * 16, 16)
            urow[j, s] = jnp.minimum(_pack_row(uidx[j, s]), HROWS - 1)
            irow[j, s] = _pack_row(iidx[j, s])

    def emit(j, g, s, acc):
        outv[pl.ds(j * CHUNK + g * 16, 16)] = acc

    _dot_chunks(uidx, iidx, urow, irow, uf0_hbm, if_hbm, ubuf, ibuf,
                (sem_u0, sem_u1), (sem_i0, sem_i1), emit)

    pltpu.sync_copy(outv, out_hbm.at[pl.ds(base, BPW)])


@functools.partial(
    pl.kernel, mesh=_mesh,
    out_type=jax.ShapeDtypeStruct((B,), jnp.float32),
    compiler_params=_sc_params,
    scratch_types=_scratch + [pltpu.VMEM((BPW,), jnp.float32)],
)
def _mf_pass2(user_hbm, item_hbm, uf1_hbm, if_hbm, part_hbm, out_hbm,
              uidx, iidx, urow, irow, ubuf, ibuf, outv,
              sem_u0, sem_u1, sem_i0, sem_i1, partv):
    wid = lax.axis_index("s") * NC + lax.axis_index("c")
    base = wid * BPW

    idx_copies = [pltpu.async_copy(
        part_hbm.at[pl.ds(base, BPW)], partv, sem_i1)]
    for j in range(NCH):
        idx_copies.append(pltpu.async_copy(
            user_hbm.at[pl.ds(base + j * CHUNK, CHUNK)], uidx.at[j], sem_u0))
        idx_copies.append(pltpu.async_copy(
            item_hbm.at[pl.ds(base + j * CHUNK, CHUNK)], iidx.at[j], sem_i0))
    for c in idx_copies:
        c.wait()
    for j in range(NCH):
        for g in range(GRP):
            s = pl.ds(g * 16, 16)
            urow[j, s] = jnp.maximum(_pack_row(uidx[j, s]) - HROWS, 0)
            irow[j, s] = _pack_row(iidx[j, s])

    def emit(j, g, s, acc):
        o = pl.ds(j * CHUNK + g * 16, 16)
        outv[o] = jnp.where(uidx[j, s] >= SPLIT, acc, partv[o])

    _dot_chunks(uidx, iidx, urow, irow, uf1_hbm, if_hbm, ubuf, ibuf,
                (sem_u0, sem_u1), (sem_i0, sem_i1), emit)

    pltpu.sync_copy(outv, out_hbm.at[pl.ds(base, BPW)])


def kernel(user, item, user_factors, item_factors):
    uft = user_factors.T
    if128 = _repack(item_factors.T, 0, (N_I + BLK - 1) // BLK)
    uf0 = _repack(uft, 0, HBLKS)
    part = _mf_pass1(user, item, uf0, if128)
    uf1 = _repack(uft, HBLKS, UBLKS - HBLKS)
    return _mf_pass2(user, item, uf1, if128, part)


# speedup vs baseline: 5.7990x; 5.7990x over previous
import functools

import jax
import jax.numpy as jnp
from jax import lax
from jax.experimental import pallas as pl
from jax.experimental.pallas import tpu as pltpu
from jax.experimental.pallas import tpu_sc as plsc

B = 16384
F = 32
NC = 2
NS = 16
NW = NC * NS
BPW = B // NW
CHUNK = 128
NCH = BPW // CHUNK
GRP = CHUNK // 16

BLK = 65536


QTR = BLK // 4


def _repack_body(src_ref, dst_ref):
    for s in range(BLK // 512):
        z = jnp.concatenate(
            [src_ref[:, pl.ds(512 * s + 128 * g, 128)] for g in range(4)],
            axis=0)
        dst_ref[pl.ds(s * 128, 128), :] = jnp.transpose(z)


def _repack(table_t):
    f, n = table_t.shape
    grid = (n + BLK - 1) // BLK
    return pl.pallas_call(
        _repack_body,
        grid=(grid,),
        in_specs=[pl.BlockSpec((F, BLK), lambda i: (0, i))],
        out_specs=pl.BlockSpec((QTR, 128), lambda i: (i, 0)),
        out_shape=jax.ShapeDtypeStruct((grid * QTR, 128), jnp.float32),
    )(table_t)


_mesh = plsc.VectorSubcoreMesh(core_axis_name="c", subcore_axis_name="s")


@functools.partial(
    pl.kernel,
    mesh=_mesh,
    out_type=jax.ShapeDtypeStruct((B,), jnp.float32),
    compiler_params=pltpu.CompilerParams(needs_layout_passes=False),
    scratch_types=[
        pltpu.VMEM((NCH, CHUNK), jnp.int32),
        pltpu.VMEM((NCH, CHUNK), jnp.int32),
        pltpu.VMEM((NCH, CHUNK), jnp.int32),
        pltpu.VMEM((NCH, CHUNK), jnp.int32),
        pltpu.VMEM((2, CHUNK, 128), jnp.float32),
        pltpu.VMEM((2, CHUNK, 128), jnp.float32),
        pltpu.VMEM((BPW,), jnp.float32),
        pltpu.SemaphoreType.DMA,
        pltpu.SemaphoreType.DMA,
        pltpu.SemaphoreType.DMA,
        pltpu.SemaphoreType.DMA,
    ],
)
def _mf_sc(user_hbm, item_hbm, uf_hbm, if_hbm, out_hbm,
           uidx, iidx, urow, irow, ubuf, ibuf, outv,
           sem_u0, sem_u1, sem_i0, sem_i1):
    wid = lax.axis_index("s") * NC + lax.axis_index("c")
    base = wid * BPW

    idx_copies = []
    for j in range(NCH):
        idx_copies.append(pltpu.async_copy(
            user_hbm.at[pl.ds(base + j * CHUNK, CHUNK)], uidx.at[j], sem_u0))
        idx_copies.append(pltpu.async_copy(
            item_hbm.at[pl.ds(base + j * CHUNK, CHUNK)], iidx.at[j], sem_i0))
    for c in idx_copies:
        c.wait()
    for j in range(NCH):
        for g in range(GRP):
            s = pl.ds(g * 16, 16)
            u = uidx[j, s]
            i = iidx[j, s]
            urow[j, s] = lax.shift_left(
                lax.shift_right_logical(u, 9), 7) + jnp.bitwise_and(u, 127)
            irow[j, s] = lax.shift_left(
                lax.shift_right_logical(i, 9), 7) + jnp.bitwise_and(i, 127)

    sems_u = (sem_u0, sem_u1)
    sems_i = (sem_i0, sem_i1)

    def fire(j):
        p = j % 2
        cu = pltpu.async_copy(uf_hbm.at[urow.at[j]], ubuf.at[p], sems_u[p])
        ci = pltpu.async_copy(if_hbm.at[irow.at[j]], ibuf.at[p], sems_i[p])
        return cu, ci

    pending = fire(0)
    for j in range(NCH):
        nxt = fire(j + 1) if j + 1 < NCH else None
        cu, ci = pending
        cu.wait()
        ci.wait()
        pending = nxt
        p = j % 2

        def body(g, carry):
            rows = g * 16 + lax.iota(jnp.int32, 16)
            s = pl.ds(g * 16, 16)
            ucol = lax.shift_left(
                jnp.bitwise_and(lax.shift_right_logical(uidx[j, s], 7), 3), 5)
            icol = lax.shift_left(
                jnp.bitwise_and(lax.shift_right_logical(iidx[j, s], 7), 3), 5)
            acc = jnp.zeros((16,), jnp.float32)
            for f in range(F):
                gu = plsc.load_gather(ubuf.at[p], [rows, ucol + f])
                gi = plsc.load_gather(ibuf.at[p], [rows, icol + f])
                acc = acc + gu * gi
            outv[pl.ds(j * CHUNK + g * 16, 16)] = acc
            return carry

        lax.fori_loop(0, GRP, body, 0)

    pltpu.sync_copy(outv, out_hbm.at[pl.ds(base, BPW)])


def kernel(user, item, user_factors, item_factors):
    uf128 = _repack(user_factors.T)
    if128 = _repack(item_factors.T)
    return _mf_sc(user, item, uf128, if128)
